# baseline (device time: 21308 ns/iter reference)
import jax
import jax.numpy as jnp
from jax import lax
from jax.experimental import pallas as pl
from jax.experimental.pallas import tpu as pltpu

BLK_ROWS = 512


def kernel(x, dy, gamma):
    m, d = x.shape
    grid = m // BLK_ROWS

    def body(x_ref, dy_ref, out_ref, acc_ref, comm_ref, send_sem, recv_sem):
        step = pl.program_id(0)
        my_x = lax.axis_index("x")
        my_y = lax.axis_index("y")
        peer = (my_x, 1 - my_y)

        @pl.when(step == 0)
        def _():
            barrier = pltpu.get_barrier_semaphore()
            pl.semaphore_signal(
                barrier, inc=1, device_id=peer,
                device_id_type=pl.DeviceIdType.MESH,
            )
            pl.semaphore_wait(barrier, 1)
            acc_ref[...] = jnp.zeros_like(acc_ref)

        xb = x_ref[...]
        dyb = dy_ref[...]
        ones_col = jnp.ones((d, 1), jnp.float32)
        inv_d = 1.0 / d
        mu = jnp.dot(xb, ones_col, preferred_element_type=jnp.float32) * inv_d
        xx = xb * xb
        ex2 = jnp.dot(xx, ones_col, preferred_element_type=jnp.float32) * inv_d
        rstd = lax.rsqrt(ex2 - mu * mu + 1e-5)
        b = dyb * (xb * rstd - mu * rstd)
        ones_row = jnp.ones((1, BLK_ROWS), jnp.float32)
        dgamma = jnp.dot(ones_row, b, preferred_element_type=jnp.float32)
        dbeta = jnp.dot(ones_row, dyb, preferred_element_type=jnp.float32)
        acc_ref[...] += jnp.concatenate([dgamma, dbeta], axis=0)

        @pl.when(step == grid - 1)
        def _():
            rdma = pltpu.make_async_remote_copy(
                src_ref=acc_ref,
                dst_ref=comm_ref,
                send_sem=send_sem,
                recv_sem=recv_sem,
                device_id=peer,
                device_id_type=pl.DeviceIdType.MESH,
            )
            rdma.start()
            rdma.wait()
            out_ref[...] = acc_ref[...] + comm_ref[...]

    return pl.pallas_call(
        body,
        grid=(grid,),
        out_shape=jax.ShapeDtypeStruct((2, d), jnp.float32),
        in_specs=[
            pl.BlockSpec((BLK_ROWS, d), lambda i: (i, 0)),
            pl.BlockSpec((BLK_ROWS, d), lambda i: (i, 0)),
        ],
        out_specs=pl.BlockSpec((2, d), lambda i: (0, 0)),
        scratch_shapes=[
            pltpu.VMEM((2, d), jnp.float32),
            pltpu.VMEM((2, d), jnp.float32),
            pltpu.SemaphoreType.DMA,
            pltpu.SemaphoreType.DMA,
        ],
        compiler_params=pltpu.CompilerParams(collective_id=0),
    )(x, dy)


# device time: 16883 ns/iter; 1.2621x vs baseline; 1.2621x over previous
import jax
import jax.numpy as jnp
from jax import lax
from jax.experimental import pallas as pl
from jax.experimental.pallas import tpu as pltpu

BLK_ROWS = 512


def kernel(x, dy, gamma):
    m, d = x.shape
    half = m // 2
    grid = half // BLK_ROWS

    def body(off_ref, x_ref, dy_ref, out_ref, acc_ref, comm_ref,
             send_sems, recv_sems):
        step = pl.program_id(0)
        my_x = lax.axis_index("x")
        my_y = lax.axis_index("y")
        peers = [(my_x, 1 - my_y), (1 - my_x, my_y), (1 - my_x, 1 - my_y)]

        @pl.when(step == 0)
        def _():
            barrier = pltpu.get_barrier_semaphore()
            for p in peers:
                pl.semaphore_signal(
                    barrier, inc=1, device_id=p,
                    device_id_type=pl.DeviceIdType.MESH,
                )
            pl.semaphore_wait(barrier, 3)
            acc_ref[...] = jnp.zeros_like(acc_ref)

        xb = x_ref[...]
        dyb = dy_ref[...]
        mu = jnp.mean(xb, axis=1, keepdims=True)
        ex2 = jnp.mean(xb * xb, axis=1, keepdims=True)
        rstd = lax.rsqrt(ex2 - mu * mu + 1e-5)
        dgamma = jnp.sum(dyb * (xb * rstd - mu * rstd), axis=0, keepdims=True)
        dbeta = jnp.sum(dyb, axis=0, keepdims=True)
        acc_ref[...] += jnp.concatenate([dgamma, dbeta], axis=0)

        @pl.when(step == grid - 1)
        def _():
            rdmas = []
            for slot, p in enumerate(peers):
                r = pltpu.make_async_remote_copy(
                    src_ref=acc_ref,
                    dst_ref=comm_ref.at[slot],
                    send_sem=send_sems.at[slot],
                    recv_sem=recv_sems.at[slot],
                    device_id=p,
                    device_id_type=pl.DeviceIdType.MESH,
                )
                r.start()
                rdmas.append(r)
            for r in rdmas:
                r.wait()
            out_ref[...] = (
                acc_ref[...] + comm_ref[0] + comm_ref[1] + comm_ref[2]
            )

    off = (lax.axis_index("x") * grid).astype(jnp.int32).reshape((1,))
    grid_spec = pltpu.PrefetchScalarGridSpec(
        num_scalar_prefetch=1,
        grid=(grid,),
        in_specs=[
            pl.BlockSpec((BLK_ROWS, d), lambda i, off: (off[0] + i, 0)),
            pl.BlockSpec((BLK_ROWS, d), lambda i, off: (off[0] + i, 0)),
        ],
        out_specs=pl.BlockSpec((2, d), lambda i, off: (0, 0)),
        scratch_shapes=[
            pltpu.VMEM((2, d), jnp.float32),
            pltpu.VMEM((3, 2, d), jnp.float32),
            pltpu.SemaphoreType.DMA((3,)),
            pltpu.SemaphoreType.DMA((3,)),
        ],
    )
    return pl.pallas_call(
        body,
        grid_spec=grid_spec,
        out_shape=jax.ShapeDtypeStruct((2, d), jnp.float32),
        compiler_params=pltpu.CompilerParams(collective_id=0),
    )(off, x, dy)
